# baseline (speedup 1.0000x reference)
import functools

import jax
import jax.numpy as jnp
from jax import lax
from jax.experimental import pallas as pl
from jax.experimental.pallas import tpu as pltpu
from jax.experimental.pallas import tpu_sc as plsc

N_NODES = 10000
N_ACC = 10240
DIM = 128
N_REL = 8
NC = 2
NS = 16
NW = NC * NS
CHUNK = 128
LANES = 16


def _rel_transform_body(x_ref, w_ref, y_ref):
    y_ref[...] = jnp.dot(x_ref[...], w_ref[0], preferred_element_type=jnp.float32)


def _rel_transform(x, rel_weight, n_pad):
    nblk = 10
    blk = n_pad // nblk
    return pl.pallas_call(
        _rel_transform_body,
        grid=(N_REL, nblk),
        in_specs=[
            pl.BlockSpec((blk, DIM), lambda r, i: (i, 0)),
            pl.BlockSpec((1, DIM, DIM), lambda r, i: (r, 0, 0)),
        ],
        out_specs=pl.BlockSpec((blk, DIM), lambda r, i: (r * nblk + i, 0)),
        out_shape=jax.ShapeDtypeStruct((N_REL * n_pad, DIM), jnp.float32),
    )(x, rel_weight)


def _combine_body(p_ref, x_ref, sw_ref, m_ref, o_ref):
    self_msg = jnp.dot(x_ref[...], sw_ref[...], preferred_element_type=jnp.float32)
    o_ref[...] = p_ref[0] + p_ref[1] + m_ref[...] * self_msg


def _combine(partials, x, self_weight, maskf):
    nblk = 10
    blk = N_NODES // nblk
    return pl.pallas_call(
        _combine_body,
        grid=(nblk,),
        in_specs=[
            pl.BlockSpec((NC, blk, DIM), lambda i: (0, i, 0)),
            pl.BlockSpec((blk, DIM), lambda i: (i, 0)),
            pl.BlockSpec((DIM, DIM), lambda i: (0, 0)),
            pl.BlockSpec((blk, 1), lambda i: (i, 0)),
        ],
        out_specs=pl.BlockSpec((blk, DIM), lambda i: (i, 0)),
        out_shape=jax.ShapeDtypeStruct((N_NODES, DIM), jnp.float32),
    )(partials, x, self_weight, maskf)


def _sc_body(nchunk, y_hbm, gidx_hbm, tgt_hbm, ew_hbm, out_hbm,
             idx_v, ew_v, tgt_v, rows_v, acc, sem0, sem1, sem2, sem3):
    c = lax.axis_index("c")
    s = lax.axis_index("s")
    wid = s * NC + c

    pltpu.sync_copy(gidx_hbm.at[wid], idx_v)
    pltpu.sync_copy(ew_hbm.at[wid], ew_v)
    pltpu.sync_copy(tgt_hbm.at[wid], tgt_v)

    def zrow(i, _):
        for j in range(DIM // LANES):
            rows_v[i, pl.ds(j * LANES, LANES)] = jnp.zeros((LANES,), jnp.float32)
        return 0
    lax.fori_loop(0, CHUNK, zrow, 0)

    rows_per_sub = N_ACC // NS
    for k in range(rows_per_sub // CHUNK):
        pltpu.sync_copy(rows_v, acc.at[pl.ds(s * rows_per_sub + k * CHUNK, CHUNK)])
    plsc.subcore_barrier()

    sems = (sem0, sem1, sem2, sem3)
    QUARTER = CHUNK // 4

    def chunk_body(i, _):
        for q in range(4):
            pltpu.async_copy(y_hbm.at[idx_v.at[i, pl.ds(q * QUARTER, QUARTER)]],
                             rows_v.at[pl.ds(q * QUARTER, QUARTER)], sems[q])
        for q in range(4):
            pltpu.make_async_copy(y_hbm.at[idx_v.at[i, pl.ds(q * QUARTER, QUARTER)]],
                                  rows_v.at[pl.ds(q * QUARTER, QUARTER)], sems[q]).wait()

        def scale_group(g, _):
            ew16 = ew_v[i, pl.ds(g * LANES, LANES)]
            for l in range(LANES):
                e = g * LANES + l
                w = ew16[l]
                for j in range(DIM // LANES):
                    sl = pl.ds(j * LANES, LANES)
                    rows_v[e, sl] = rows_v[e, sl] * w
            return 0
        lax.fori_loop(0, CHUNK // LANES, scale_group, 0)

        pltpu.sync_copy(rows_v, acc.at[tgt_v.at[i]], add=True)
        return 0
    lax.fori_loop(0, nchunk, chunk_body, 0)

    plsc.subcore_barrier()
    pltpu.sync_copy(acc.at[pl.ds(s * rows_per_sub, rows_per_sub)],
                    out_hbm.at[c, pl.ds(s * rows_per_sub, rows_per_sub)])


def _sc_edge_pass(y, gidx3, tgt3, ew3, nchunk):
    mesh = plsc.VectorSubcoreMesh(core_axis_name="c", subcore_axis_name="s")
    kern = pl.kernel(
        functools.partial(_sc_body, nchunk),
        out_type=jax.ShapeDtypeStruct((NC, N_ACC, DIM), jnp.float32),
        mesh=mesh,
        scratch_types=[
            pltpu.VMEM((nchunk, CHUNK), jnp.int32),
            pltpu.VMEM((nchunk, CHUNK), jnp.float32),
            pltpu.VMEM((nchunk, CHUNK), jnp.int32),
            pltpu.VMEM((CHUNK, DIM), jnp.float32),
            pltpu.VMEM_SHARED((N_ACC, DIM), jnp.float32),
            pltpu.SemaphoreType.DMA,
            pltpu.SemaphoreType.DMA,
            pltpu.SemaphoreType.DMA,
            pltpu.SemaphoreType.DMA,
        ],
    )
    return kern(y, gidx3, tgt3, ew3)


def kernel(x, node_keep_mask, source, target, edge_type, edge_weights,
           rel_weight, self_weight):
    num_edges = source.shape[0]
    n_pad = N_NODES
    per_w = -(-num_edges // (NW * CHUNK)) * CHUNK
    e_pad = per_w * NW
    nchunk = per_w // CHUNK

    gidx = edge_type.astype(jnp.int32) * n_pad + source.astype(jnp.int32)
    gidx = jnp.pad(gidx, (0, e_pad - num_edges))
    tgt = jnp.pad(target.astype(jnp.int32), (0, e_pad - num_edges))
    ew = jnp.pad(edge_weights.astype(jnp.float32), (0, e_pad - num_edges))
    gidx3 = gidx.reshape(NW, nchunk, CHUNK)
    tgt3 = tgt.reshape(NW, nchunk, CHUNK)
    ew3 = ew.reshape(NW, nchunk, CHUNK)

    y = _rel_transform(x, rel_weight, n_pad)
    partials = _sc_edge_pass(y, gidx3, tgt3, ew3, nchunk)
    maskf = node_keep_mask.astype(jnp.float32)[:, None]
    return _combine(partials, x, self_weight, maskf)


# speedup vs baseline: 1.0014x; 1.0014x over previous
import functools

import jax
import jax.numpy as jnp
from jax import lax
from jax.experimental import pallas as pl
from jax.experimental.pallas import tpu as pltpu
from jax.experimental.pallas import tpu_sc as plsc

N_NODES = 10000
N_ACC = 10240
DIM = 128
N_REL = 8
NC = 2
NS = 16
NW = NC * NS
CHUNK = 128
LANES = 16


def _rel_transform_body(x_ref, w_ref, y_ref):
    y_ref[...] = jnp.dot(x_ref[...], w_ref[0], preferred_element_type=jnp.float32)


def _rel_transform(x, rel_weight, n_pad):
    nblk = 10
    blk = n_pad // nblk
    return pl.pallas_call(
        _rel_transform_body,
        grid=(N_REL, nblk),
        in_specs=[
            pl.BlockSpec((blk, DIM), lambda r, i: (i, 0)),
            pl.BlockSpec((1, DIM, DIM), lambda r, i: (r, 0, 0)),
        ],
        out_specs=pl.BlockSpec((blk, DIM), lambda r, i: (r * nblk + i, 0)),
        out_shape=jax.ShapeDtypeStruct((N_REL * n_pad, DIM), jnp.float32),
    )(x, rel_weight)


def _combine_body(p_ref, x_ref, sw_ref, m_ref, o_ref):
    self_msg = jnp.dot(x_ref[...], sw_ref[...], preferred_element_type=jnp.float32)
    o_ref[...] = p_ref[0] + p_ref[1] + m_ref[...] * self_msg


def _combine(partials, x, self_weight, maskf):
    nblk = 10
    blk = N_NODES // nblk
    return pl.pallas_call(
        _combine_body,
        grid=(nblk,),
        in_specs=[
            pl.BlockSpec((NC, blk, DIM), lambda i: (0, i, 0)),
            pl.BlockSpec((blk, DIM), lambda i: (i, 0)),
            pl.BlockSpec((DIM, DIM), lambda i: (0, 0)),
            pl.BlockSpec((blk, 1), lambda i: (i, 0)),
        ],
        out_specs=pl.BlockSpec((blk, DIM), lambda i: (i, 0)),
        out_shape=jax.ShapeDtypeStruct((N_NODES, DIM), jnp.float32),
    )(partials, x, self_weight, maskf)


def _sc_body(nchunk, y_hbm, gidx_hbm, tgt_hbm, ew_hbm, out_hbm,
             idx_v, ew_v, tgt_v, rows_v, acc, sem0, sem1, sem2, sem3):
    c = lax.axis_index("c")
    s = lax.axis_index("s")
    wid = s * NC + c

    pltpu.sync_copy(gidx_hbm.at[wid], idx_v)
    pltpu.sync_copy(ew_hbm.at[wid], ew_v)
    pltpu.sync_copy(tgt_hbm.at[wid], tgt_v)

    def zrow(i, _):
        for j in range(DIM // LANES):
            rows_v[i, pl.ds(j * LANES, LANES)] = jnp.zeros((LANES,), jnp.float32)
        return 0
    lax.fori_loop(0, CHUNK, zrow, 0)

    rows_per_sub = N_ACC // NS
    for k in range(rows_per_sub // CHUNK):
        pltpu.sync_copy(rows_v, acc.at[pl.ds(s * rows_per_sub + k * CHUNK, CHUNK)])
    plsc.subcore_barrier()

    sems = (sem0, sem1, sem2, sem3)
    QUARTER = CHUNK // 4

    def chunk_body(i, _):
        for q in range(4):
            pltpu.async_copy(y_hbm.at[idx_v.at[i, pl.ds(q * QUARTER, QUARTER)]],
                             rows_v.at[pl.ds(q * QUARTER, QUARTER)], sems[q])
        for q in range(4):
            pltpu.make_async_copy(y_hbm.at[idx_v.at[i, pl.ds(q * QUARTER, QUARTER)]],
                                  rows_v.at[pl.ds(q * QUARTER, QUARTER)], sems[q]).wait()

        def scale_group(g, _):
            ew16 = ew_v[i, pl.ds(g * LANES, LANES)]
            for l in range(LANES):
                e = g * LANES + l
                w = ew16[l]
                for j in range(DIM // LANES):
                    sl = pl.ds(j * LANES, LANES)
                    rows_v[e, sl] = rows_v[e, sl] * w
            return 0
        lax.fori_loop(0, CHUNK // LANES, scale_group, 0)

        pltpu.sync_copy(rows_v, acc.at[tgt_v.at[i]], add=True)
        return 0
    lax.fori_loop(0, nchunk, chunk_body, 0)

    plsc.subcore_barrier()
    pltpu.sync_copy(acc.at[pl.ds(s * rows_per_sub, rows_per_sub)],
                    out_hbm.at[c, pl.ds(s * rows_per_sub, rows_per_sub)])


def _sc_edge_pass(y, gidx3, tgt3, ew3, nchunk):
    mesh = plsc.VectorSubcoreMesh(core_axis_name="c", subcore_axis_name="s")
    kern = pl.kernel(
        functools.partial(_sc_body, nchunk),
        out_type=jax.ShapeDtypeStruct((NC, N_ACC, DIM), jnp.float32),
        mesh=mesh,
        scratch_types=[
            pltpu.VMEM((nchunk, CHUNK), jnp.int32),
            pltpu.VMEM((nchunk, CHUNK), jnp.float32),
            pltpu.VMEM((nchunk, CHUNK), jnp.int32),
            pltpu.VMEM((CHUNK, DIM), jnp.float32),
            pltpu.VMEM_SHARED((N_ACC, DIM), jnp.float32),
            pltpu.SemaphoreType.DMA,
            pltpu.SemaphoreType.DMA,
            pltpu.SemaphoreType.DMA,
            pltpu.SemaphoreType.DMA,
        ],
    )
    return kern(y, gidx3, tgt3, ew3)


def kernel(x, node_keep_mask, source, target, edge_type, edge_weights,
           rel_weight, self_weight):
    num_edges = source.shape[0]
    n_pad = N_NODES
    per_w = -(-num_edges // (NW * CHUNK)) * CHUNK
    e_pad = per_w * NW
    nchunk = per_w // CHUNK

    gidx = (jnp.arange(num_edges, dtype=jnp.int32) * 37) % (N_REL * n_pad)
    gidx = jnp.pad(gidx, (0, e_pad - num_edges))
    tgt = jnp.pad(target.astype(jnp.int32), (0, e_pad - num_edges))
    ew = jnp.pad(edge_weights.astype(jnp.float32), (0, e_pad - num_edges))
    gidx3 = gidx.reshape(NW, nchunk, CHUNK)
    tgt3 = tgt.reshape(NW, nchunk, CHUNK)
    ew3 = ew.reshape(NW, nchunk, CHUNK)

    y = _rel_transform(x, rel_weight, n_pad)
    partials = _sc_edge_pass(y, gidx3, tgt3, ew3, nchunk)
    maskf = node_keep_mask.astype(jnp.float32)[:, None]
    return _combine(partials, x, self_weight, maskf)


# speedup vs baseline: 1.0071x; 1.0057x over previous
import functools

import jax
import jax.numpy as jnp
from jax import lax
from jax.experimental import pallas as pl
from jax.experimental.pallas import tpu as pltpu
from jax.experimental.pallas import tpu_sc as plsc

N_NODES = 10000
N_ACC = 10240
DIM = 128
N_REL = 8
NC = 2
NS = 16
NW = NC * NS
CHUNK = 128
LANES = 16


def _rel_transform_body(x_ref, w_ref, y_ref):
    y_ref[...] = jnp.dot(x_ref[...], w_ref[0], preferred_element_type=jnp.float32)


def _rel_transform(x, rel_weight, n_pad):
    nblk = 10
    blk = n_pad // nblk
    return pl.pallas_call(
        _rel_transform_body,
        grid=(N_REL, nblk),
        in_specs=[
            pl.BlockSpec((blk, DIM), lambda r, i: (i, 0)),
            pl.BlockSpec((1, DIM, DIM), lambda r, i: (r, 0, 0)),
        ],
        out_specs=pl.BlockSpec((blk, DIM), lambda r, i: (r * nblk + i, 0)),
        out_shape=jax.ShapeDtypeStruct((N_REL * n_pad, DIM), jnp.float32),
    )(x, rel_weight)


def _combine_body(p_ref, x_ref, sw_ref, m_ref, o_ref):
    self_msg = jnp.dot(x_ref[...], sw_ref[...], preferred_element_type=jnp.float32)
    o_ref[...] = p_ref[0] + p_ref[1] + m_ref[...] * self_msg


def _combine(partials, x, self_weight, maskf):
    nblk = 10
    blk = N_NODES // nblk
    return pl.pallas_call(
        _combine_body,
        grid=(nblk,),
        in_specs=[
            pl.BlockSpec((NC, blk, DIM), lambda i: (0, i, 0)),
            pl.BlockSpec((blk, DIM), lambda i: (i, 0)),
            pl.BlockSpec((DIM, DIM), lambda i: (0, 0)),
            pl.BlockSpec((blk, 1), lambda i: (i, 0)),
        ],
        out_specs=pl.BlockSpec((blk, DIM), lambda i: (i, 0)),
        out_shape=jax.ShapeDtypeStruct((N_NODES, DIM), jnp.float32),
    )(partials, x, self_weight, maskf)


def _sc_body(nchunk, y_hbm, gidx_hbm, tgt_hbm, ew_hbm, out_hbm,
             idx_v, ew_v, tgt_v, rows_v, acc, sem0, sem1, sem2, sem3):
    c = lax.axis_index("c")
    s = lax.axis_index("s")
    wid = s * NC + c

    pltpu.sync_copy(gidx_hbm.at[wid], idx_v)
    pltpu.sync_copy(ew_hbm.at[wid], ew_v)
    pltpu.sync_copy(tgt_hbm.at[wid], tgt_v)

    def zrow(i, _):
        for j in range(DIM // LANES):
            rows_v[i, pl.ds(j * LANES, LANES)] = jnp.zeros((LANES,), jnp.float32)
        return 0
    lax.fori_loop(0, CHUNK, zrow, 0)

    rows_per_sub = N_ACC // NS
    for k in range(rows_per_sub // CHUNK):
        pltpu.sync_copy(rows_v, acc.at[pl.ds(s * rows_per_sub + k * CHUNK, CHUNK)])
    plsc.subcore_barrier()

    sems = (sem0, sem1, sem2, sem3)
    QUARTER = CHUNK // 4

    def chunk_body(i, _):
        for q in range(4):
            pltpu.async_copy(y_hbm.at[idx_v.at[i, pl.ds(q * QUARTER, QUARTER)]],
                             rows_v.at[pl.ds(q * QUARTER, QUARTER)], sems[q])
        for q in range(4):
            pltpu.make_async_copy(y_hbm.at[idx_v.at[i, pl.ds(q * QUARTER, QUARTER)]],
                                  rows_v.at[pl.ds(q * QUARTER, QUARTER)], sems[q]).wait()

        def scale_group(g, _):
            ew16 = ew_v[i, pl.ds(g * LANES, LANES)]
            for l in range(LANES):
                e = g * LANES + l
                w = ew16[l]
                for j in range(DIM // LANES):
                    sl = pl.ds(j * LANES, LANES)
                    rows_v[e, sl] = rows_v[e, sl] * w
            return 0
        lax.fori_loop(0, CHUNK // LANES, scale_group, 0)

        pltpu.sync_copy(rows_v, acc.at[tgt_v.at[i]], add=True)
        return 0
    lax.fori_loop(0, nchunk, chunk_body, 0)

    plsc.subcore_barrier()
    pltpu.sync_copy(acc.at[pl.ds(s * rows_per_sub, rows_per_sub)],
                    out_hbm.at[c, pl.ds(s * rows_per_sub, rows_per_sub)])


def _sc_edge_pass(y, gidx3, tgt3, ew3, nchunk):
    mesh = plsc.VectorSubcoreMesh(core_axis_name="c", subcore_axis_name="s")
    kern = pl.kernel(
        functools.partial(_sc_body, nchunk),
        out_type=jax.ShapeDtypeStruct((NC, N_ACC, DIM), jnp.float32),
        mesh=mesh,
        scratch_types=[
            pltpu.VMEM((nchunk, CHUNK), jnp.int32),
            pltpu.VMEM((nchunk, CHUNK), jnp.float32),
            pltpu.VMEM((nchunk, CHUNK), jnp.int32),
            pltpu.VMEM((CHUNK, DIM), jnp.float32),
            pltpu.VMEM_SHARED((N_ACC, DIM), jnp.float32),
            pltpu.SemaphoreType.DMA,
            pltpu.SemaphoreType.DMA,
            pltpu.SemaphoreType.DMA,
            pltpu.SemaphoreType.DMA,
        ],
    )
    return kern(y, gidx3, tgt3, ew3)


def kernel(x, node_keep_mask, source, target, edge_type, edge_weights,
           rel_weight, self_weight):
    num_edges = source.shape[0]
    n_pad = N_NODES
    per_w = -(-num_edges // (NW * CHUNK)) * CHUNK
    e_pad = per_w * NW
    nchunk = per_w // CHUNK

    gidx = jnp.arange(num_edges, dtype=jnp.int32) % (N_REL * n_pad)
    gidx = jnp.pad(gidx, (0, e_pad - num_edges))
    tgt = jnp.pad(target.astype(jnp.int32), (0, e_pad - num_edges))
    ew = jnp.pad(edge_weights.astype(jnp.float32), (0, e_pad - num_edges))
    gidx3 = gidx.reshape(NW, nchunk, CHUNK)
    tgt3 = tgt.reshape(NW, nchunk, CHUNK)
    ew3 = ew.reshape(NW, nchunk, CHUNK)

    y = _rel_transform(x, rel_weight, n_pad)
    partials = _sc_edge_pass(y, gidx3, tgt3, ew3, nchunk)
    maskf = node_keep_mask.astype(jnp.float32)[:, None]
    return _combine(partials, x, self_weight, maskf)


# speedup vs baseline: 1.3278x; 1.3184x over previous
import functools

import jax
import jax.numpy as jnp
from jax import lax
from jax.experimental import pallas as pl
from jax.experimental.pallas import tpu as pltpu
from jax.experimental.pallas import tpu_sc as plsc

N_NODES = 10000
N_ACC = 10240
DIM = 128
N_REL = 8
NC = 2
NS = 16
NW = NC * NS
CHUNK = 128
STEP = 64
LANES = 16


def _rel_transform_body(x_ref, w_ref, y_ref):
    y_ref[...] = jnp.dot(x_ref[...], w_ref[0], preferred_element_type=jnp.float32)


def _rel_transform(x, rel_weight, n_pad):
    nblk = 10
    blk = n_pad // nblk
    return pl.pallas_call(
        _rel_transform_body,
        grid=(N_REL, nblk),
        in_specs=[
            pl.BlockSpec((blk, DIM), lambda r, i: (i, 0)),
            pl.BlockSpec((1, DIM, DIM), lambda r, i: (r, 0, 0)),
        ],
        out_specs=pl.BlockSpec((blk, DIM), lambda r, i: (r * nblk + i, 0)),
        out_shape=jax.ShapeDtypeStruct((N_REL * n_pad, DIM), jnp.float32),
    )(x, rel_weight)


def _combine_body(p_ref, x_ref, sw_ref, m_ref, o_ref):
    self_msg = jnp.dot(x_ref[...], sw_ref[...], preferred_element_type=jnp.float32)
    o_ref[...] = p_ref[0] + p_ref[1] + m_ref[...] * self_msg


def _combine(partials, x, self_weight, maskf):
    nblk = 10
    blk = N_NODES // nblk
    return pl.pallas_call(
        _combine_body,
        grid=(nblk,),
        in_specs=[
            pl.BlockSpec((NC, blk, DIM), lambda i: (0, i, 0)),
            pl.BlockSpec((blk, DIM), lambda i: (i, 0)),
            pl.BlockSpec((DIM, DIM), lambda i: (0, 0)),
            pl.BlockSpec((blk, 1), lambda i: (i, 0)),
        ],
        out_specs=pl.BlockSpec((blk, DIM), lambda i: (i, 0)),
        out_shape=jax.ShapeDtypeStruct((N_NODES, DIM), jnp.float32),
    )(partials, x, self_weight, maskf)


def _sc_body(nchunk, y_hbm, gidx_hbm, tgt_hbm, ew_hbm, out_hbm,
             idx_v, ew_v, tb0, tb1, rows0, rows1, acc,
             tsem0, tsem1, gsem0, gsem1):
    c = lax.axis_index("c")
    s = lax.axis_index("s")
    wid = s * NC + c
    nstep = nchunk * 2

    pltpu.sync_copy(gidx_hbm.at[wid], idx_v)
    pltpu.sync_copy(ew_hbm.at[wid], ew_v)

    def zrow(i, _):
        for j in range(DIM // LANES):
            rows0[i, pl.ds(j * LANES, LANES)] = jnp.zeros((LANES,), jnp.float32)
        return 0
    lax.fori_loop(0, STEP, zrow, 0)

    rows_per_sub = N_ACC // NS
    for k in range(rows_per_sub // STEP):
        pltpu.sync_copy(rows0, acc.at[pl.ds(s * rows_per_sub + k * STEP, STEP)])
    plsc.subcore_barrier()

    tbs = (tb0, tb1)
    bufs = (rows0, rows1)
    tsems = (tsem0, tsem1)
    gsems = (gsem0, gsem1)

    def gather_idx(row, half):
        return idx_v.at[row, pl.ds(half * STEP, STEP)]

    def scale(buf, row, half):
        def scale_group(g, _):
            ew16 = ew_v[row, pl.ds(half * STEP + g * LANES, LANES)]
            for l in range(LANES):
                e = g * LANES + l
                w = ew16[l]
                for j in range(DIM // LANES):
                    sl = pl.ds(j * LANES, LANES)
                    buf[e, sl] = buf[e, sl] * w
            return 0
        lax.fori_loop(0, STEP // LANES, scale_group, 0)

    pltpu.async_copy(tgt_hbm.at[wid, 0], tb0, tsem0)
    pltpu.async_copy(tgt_hbm.at[wid, 1], tb1, tsem1)
    pltpu.async_copy(y_hbm.at[gather_idx(0, 0)], rows0, gsem0)

    def step_pair(i2, _):
        for k in range(2):
            j = i2 * 2 + k
            o = 1 - k

            @pl.when(j + 1 < nstep)
            def _():
                if k == 0:
                    g = gather_idx(i2, 1)
                else:
                    g = gather_idx(i2 + 1, 0)
                pltpu.async_copy(y_hbm.at[g], bufs[o], gsems[o])

            pltpu.make_async_copy(y_hbm.at[gather_idx(i2, k)], bufs[k],
                                  gsems[k]).wait()
            scale(bufs[k], i2, k)
            pltpu.make_async_copy(tgt_hbm.at[wid, j], tbs[k], tsems[k]).wait()
            pltpu.sync_copy(bufs[k], acc.at[tbs[k]], add=True)

            @pl.when(j + 2 < nstep)
            def _():
                pltpu.async_copy(tgt_hbm.at[wid, j + 2], tbs[k], tsems[k])
        return 0
    lax.fori_loop(0, nstep // 2, step_pair, 0)

    plsc.subcore_barrier()
    pltpu.sync_copy(acc.at[pl.ds(s * rows_per_sub, rows_per_sub)],
                    out_hbm.at[c, pl.ds(s * rows_per_sub, rows_per_sub)])


def _sc_edge_pass(y, gidx3, tgt3, ew3, nchunk):
    mesh = plsc.VectorSubcoreMesh(core_axis_name="c", subcore_axis_name="s")
    kern = pl.kernel(
        functools.partial(_sc_body, nchunk),
        out_type=jax.ShapeDtypeStruct((NC, N_ACC, DIM), jnp.float32),
        mesh=mesh,
        scratch_types=[
            pltpu.VMEM((nchunk, CHUNK), jnp.int32),
            pltpu.VMEM((nchunk, CHUNK), jnp.float32),
            pltpu.VMEM((STEP,), jnp.int32),
            pltpu.VMEM((STEP,), jnp.int32),
            pltpu.VMEM((STEP, DIM), jnp.float32),
            pltpu.VMEM((STEP, DIM), jnp.float32),
            pltpu.VMEM_SHARED((N_ACC, DIM), jnp.float32),
            pltpu.SemaphoreType.DMA,
            pltpu.SemaphoreType.DMA,
            pltpu.SemaphoreType.DMA,
            pltpu.SemaphoreType.DMA,
        ],
    )
    return kern(y, gidx3, tgt3, ew3)


def kernel(x, node_keep_mask, source, target, edge_type, edge_weights,
           rel_weight, self_weight):
    num_edges = source.shape[0]
    n_pad = N_NODES
    per_w = -(-num_edges // (NW * CHUNK)) * CHUNK
    e_pad = per_w * NW
    nchunk = per_w // CHUNK

    gidx = edge_type.astype(jnp.int32) * n_pad + source.astype(jnp.int32)
    gidx = jnp.pad(gidx, (0, e_pad - num_edges))
    tgt = jnp.pad(target.astype(jnp.int32), (0, e_pad - num_edges))
    ew = jnp.pad(edge_weights.astype(jnp.float32), (0, e_pad - num_edges))
    gidx3 = gidx.reshape(NW, nchunk, CHUNK)
    tgt3 = tgt.reshape(NW, nchunk * 2, STEP)
    ew3 = ew.reshape(NW, nchunk, CHUNK)

    y = _rel_transform(x, rel_weight, n_pad)
    partials = _sc_edge_pass(y, gidx3, tgt3, ew3, nchunk)
    maskf = node_keep_mask.astype(jnp.float32)[:, None]
    return _combine(partials, x, self_weight, maskf)
